# Initial kernel scaffold; baseline (speedup 1.0000x reference)
import jax
import jax.numpy as jnp
from jax.experimental import pallas as pl


def kernel(items, samples, emb):
    raise NotImplementedError("write your pallas kernel here")


# speedup vs baseline: 2.7585x; 2.7585x over previous
import functools

import jax
import jax.numpy as jnp
from jax import lax
from jax.experimental import pallas as pl
from jax.experimental.pallas import tpu as pltpu
from jax.experimental.pallas import tpu_sc as plsc

_NC = 2
_NS = 16
_NW = _NC * _NS
_L = 16

_B = 16384
_N = 50
_D = 64
_BPW = _B // _NW
_CHUNK = 8
_NCHUNK = _BPW // _CHUNK
_SROWS = _CHUNK * _N
_GSEG = 80
_NGATH = _SROWS // _GSEG
_NPAD = 64


def _sc_body(emb_hbm, items_hbm, samples_hbm, out_hbm,
             sidx_v, iidx_v, rows_s, rows_i, out_v, sem):
    wid = lax.axis_index("s") * _NC + lax.axis_index("c")
    base_b = wid * _BPW

    pltpu.sync_copy(samples_hbm.at[pl.ds(base_b * _N, _BPW * _N)], sidx_v)
    pltpu.sync_copy(items_hbm.at[pl.ds(base_b, _BPW)], iidx_v)

    lane = lax.iota(jnp.int32, _L)

    def issue(c, buf):
        for j in range(_NGATH):
            off = c * _SROWS + j * _GSEG
            pltpu.async_copy(
                emb_hbm.at[sidx_v.at[pl.ds(off, _GSEG)]],
                rows_s.at[buf, pl.ds(j * _GSEG, _GSEG)], sem)
        pltpu.async_copy(
            emb_hbm.at[iidx_v.at[pl.ds(c * _CHUNK, _CHUNK)]],
            rows_i.at[buf], sem)

    def drain(buf):
        for j in range(_NGATH):
            pltpu.make_async_copy(
                emb_hbm.at[pl.ds(0, _GSEG)],
                rows_s.at[buf, pl.ds(j * _GSEG, _GSEG)], sem).wait()
        pltpu.make_async_copy(
            emb_hbm.at[pl.ds(0, _CHUNK)], rows_i.at[buf], sem).wait()

    issue(0, 0)

    def chunk_body(c, carry):
        buf = lax.rem(c, 2)
        drain(buf)

        @pl.when(c + 1 < _NCHUNK)
        def _():
            issue(c + 1, 1 - buf)

        def row_body(b, carry2):
            i0 = rows_i[buf, b, 0:_L]
            i1 = rows_i[buf, b, _L:2 * _L]
            i2 = rows_i[buf, b, 2 * _L:3 * _L]
            i3 = rows_i[buf, b, 3 * _L:4 * _L]
            rbase = b * _N
            wb = c * _CHUNK + b
            for g in range(_NPAD // _L):
                nj = min(_L, _N - g * _L)
                acc = jnp.zeros((_L,), jnp.float32)
                for j in range(nj):
                    r = rbase + g * _L + j
                    p = (i0 * rows_s[buf, r, 0:_L]
                         + i1 * rows_s[buf, r, _L:2 * _L]
                         + i2 * rows_s[buf, r, 2 * _L:3 * _L]
                         + i3 * rows_s[buf, r, 3 * _L:4 * _L])
                    acc = jnp.where(lane == j, jnp.sum(p), acc)
                out_v[wb, g * _L:(g + 1) * _L] = acc
            return carry2

        lax.fori_loop(0, _CHUNK, row_body, 0)
        return carry

    lax.fori_loop(0, _NCHUNK, chunk_body, 0)
    pltpu.sync_copy(out_v, out_hbm.at[pl.ds(base_b, _BPW)])


@jax.jit
def _scores_padded(emb, items_flat, samples_flat):
    mesh = plsc.VectorSubcoreMesh(core_axis_name="c", subcore_axis_name="s")
    f = functools.partial(
        pl.kernel,
        mesh=mesh,
        compiler_params=pltpu.CompilerParams(
            needs_layout_passes=False, use_tc_tiling_on_sc=False),
        out_type=jax.ShapeDtypeStruct((_B, _NPAD), jnp.float32),
        scratch_types=[
            pltpu.VMEM((_BPW * _N,), jnp.int32),
            pltpu.VMEM((_BPW,), jnp.int32),
            pltpu.VMEM((2, _SROWS, _D), jnp.float32),
            pltpu.VMEM((2, _CHUNK, _D), jnp.float32),
            pltpu.VMEM((_BPW, _NPAD), jnp.float32),
            pltpu.SemaphoreType.DMA,
        ],
    )(_sc_body)
    return f(emb, items_flat, samples_flat)


def kernel(items, samples, emb):
    items_flat = items.reshape(_B).astype(jnp.int32)
    samples_flat = samples.reshape(_B * _N).astype(jnp.int32)
    out = _scores_padded(emb, items_flat, samples_flat)
    return out[:, :_N]
